# baseline (speedup 1.0000x reference)
import functools

import jax
import jax.numpy as jnp
from jax import lax
from jax.experimental import pallas as pl
from jax.experimental.pallas import tpu as pltpu
from jax.experimental.pallas import tpu_sc as plsc

DENSITY = 0.05
SC_ROWS = 32


def _count_cmp(keys16, cand_s16, rows, n, strict):
    n_acc = 8
    accs = [jnp.zeros((rows, 256), jnp.int16) for _ in range(n_acc)]
    for c in range(n // 256):
        blk = keys16[:, c * 256:(c + 1) * 256]
        m = (blk > cand_s16) if strict else (blk >= cand_s16)
        accs[c % n_acc] = accs[c % n_acc] + m.astype(jnp.int16)
    while len(accs) > 1:
        accs = [a + b for a, b in zip(accs[::2], accs[1::2])]
    return jnp.sum(accs[0].astype(jnp.int32), axis=1, keepdims=True)


def _count_ge(keys16, cand_s16, rows, n):
    return _count_cmp(keys16, cand_s16, rows, n, strict=False)


def _kwinner_block(x_ref, o_ref, *, k):
    imin = jnp.int32(-2147483648)
    x = x_ref[...]
    rows, n = x.shape
    i = jax.lax.bitcast_convert_type(x, jnp.int32)
    v = jnp.where(i >= 0, i, jnp.bitwise_xor(jnp.bitwise_not(i), imin))

    hi = jax.lax.shift_right_arithmetic(v, 16).astype(jnp.int16)
    lo = jnp.bitwise_xor(v.astype(jnp.int16), jnp.int16(-32768))

    kk = jnp.int32(k)

    def to_s16(cand_u):
        return jnp.bitwise_xor(cand_u, jnp.int32(0x8000)).astype(jnp.int16)

    def body1(j, t_u):
        bit = jnp.left_shift(jnp.int32(1), 15 - j)
        cand_u = jnp.bitwise_or(t_u, bit)
        cnt = _count_ge(hi, to_s16(cand_u), rows, n)
        return jnp.where(cnt >= kk, cand_u, t_u)

    t_hi_u = jax.lax.fori_loop(0, 16, body1, jnp.zeros((rows, 1), jnp.int32))
    t_hi_s = to_s16(t_hi_u)

    tie = hi == t_hi_s
    c_gt = _count_cmp(hi, t_hi_s, rows, n, strict=True)
    k2 = kk - c_gt

    mlo = jnp.where(tie, lo, jnp.int16(-32768))

    def body2(j, t_u):
        bit = jnp.left_shift(jnp.int32(1), 15 - j)
        cand_u = jnp.bitwise_or(t_u, bit)
        cnt = _count_ge(mlo, to_s16(cand_u), rows, n)
        return jnp.where(cnt >= k2, cand_u, t_u)

    t_lo_u = jax.lax.fori_loop(0, 16, body2, jnp.zeros((rows, 1), jnp.int32))
    t_lo_s = to_s16(t_lo_u)

    keep = jnp.logical_or(hi > t_hi_s, jnp.logical_and(tie, lo >= t_lo_s))
    o_ref[...] = jnp.where(keep, x, 0.0)


def _tc_kwinner(x, tc_rows, full_rows=None):
    b, n = x.shape
    k = int(n * DENSITY)
    rows_per_block = 8
    grid = (tc_rows // rows_per_block,)
    out_rows = full_rows if full_rows is not None else tc_rows
    return pl.pallas_call(
        functools.partial(_kwinner_block, k=k),
        grid=grid,
        in_specs=[pl.BlockSpec((rows_per_block, n), lambda i: (i, 0))],
        out_specs=pl.BlockSpec((rows_per_block, n), lambda i: (i, 0)),
        out_shape=jax.ShapeDtypeStruct((out_rows, n), x.dtype),
        compiler_params=pltpu.CompilerParams(
            dimension_semantics=("parallel",)),
    )(x)


def _sc_body(x_hbm, out_hbm, xrow, key, hist, *, row_start, n_rows, n, k):
    nc = 2
    wid = lax.axis_index("s") * nc + lax.axis_index("c")
    rows_per_worker = n_rows // 32
    chunks = n // 16
    k_cap = jnp.int32(k)

    imin = jnp.int32(-2147483648)
    lane = lax.iota(jnp.int32, 16)
    lane_base = lane * 257
    ones16 = jnp.ones((16,), jnp.int32)
    zeros16 = jnp.zeros((16,), jnp.int32)

    def zero_hist():
        def zb(j, c):
            hist[pl.ds(j * 16, 16)] = zeros16
            return c
        lax.fori_loop(0, 2 * 257, zb, 0)

    def process_row(row):
        pltpu.sync_copy(x_hbm.at[row_start + row], xrow)

        c_above = jnp.int32(0)
        prefix = jnp.int32(0)

        for level in range(4):
            shift = 24 - 8 * level
            zero_hist()

            if level == 0:
                def scat0(i, c):
                    xv = xrow[pl.ds(i * 16, 16)]
                    iv = lax.bitcast_convert_type(xv, jnp.int32)
                    kv = jnp.where(iv >= 0,
                                   jnp.bitwise_xor(iv, imin),
                                   jnp.bitwise_not(iv))
                    key[pl.ds(i * 16, 16)] = kv
                    copy = jnp.bitwise_and(i, 1) * 4112
                    idx = lax.shift_right_logical(
                        kv, jnp.int32(24)) + lane_base + copy
                    plsc.addupdate_scatter(hist, [idx], ones16)
                    return c
                lax.fori_loop(0, chunks, scat0, 0, unroll=8)
            else:
                def scat(i, c, shift=shift, prefix=prefix):
                    kv = key[pl.ds(i * 16, 16)]
                    digit = jnp.bitwise_and(
                        lax.shift_right_logical(kv, jnp.int32(shift)),
                        jnp.int32(0xFF))
                    act = lax.shift_right_logical(
                        kv, jnp.int32(shift + 8)) == prefix
                    copy = jnp.bitwise_and(i, 1) * 4112
                    idx = (jnp.where(act, digit, jnp.int32(256))
                           + lane_base + copy)
                    plsc.addupdate_scatter(hist, [idx], ones16)
                    return c
                lax.fori_loop(0, chunks, scat, 0, unroll=8)

            group_counts = []
            for g in range(16):
                acc = hist[pl.ds(g * 16, 16)]
                for cp in range(2):
                    for l in range(16):
                        if cp == 0 and l == 0:
                            continue
                        acc = acc + hist[
                            pl.ds(cp * 4112 + l * 257 + g * 16, 16)]
                group_counts.append(acc)
            totals = [jnp.sum(cg) for cg in group_counts]

            found = jnp.int32(0)
            gsel = jnp.int32(0)
            above_at_sel = c_above
            run = c_above
            for g in reversed(range(16)):
                newrun = run + totals[g]
                hit = jnp.logical_and(newrun >= k_cap, found == 0)
                gsel = jnp.where(hit, jnp.int32(g), gsel)
                above_at_sel = jnp.where(hit, run, above_at_sel)
                found = jnp.where(hit, jnp.int32(1), found)
                run = newrun

            csel = group_counts[0]
            for g in range(1, 16):
                csel = jnp.where(gsel == g, group_counts[g], csel)

            suf = lax.rev(jnp.cumsum(lax.rev(csel, (0,))), (0,))
            qual = (above_at_sel + suf) >= k_cap
            n_true = jnp.sum(qual.astype(jnp.int32))
            byte = n_true - 1
            t_byte = gsel * 16 + byte
            c_above = above_at_sel + jnp.sum(
                jnp.where(lane > byte, csel, 0))
            prefix = lax.shift_left(prefix, jnp.int32(8)) + t_byte

        t_s = jnp.bitwise_xor(prefix, imin)

        def maskbody(i, c):
            kv = key[pl.ds(i * 16, 16)]
            ks = jnp.bitwise_xor(kv, imin)
            xv = xrow[pl.ds(i * 16, 16)]
            xrow[pl.ds(i * 16, 16)] = jnp.where(ks >= t_s, xv, 0.0)
            return c
        lax.fori_loop(0, chunks, maskbody, 0, unroll=8)

        pltpu.sync_copy(xrow, out_hbm.at[row])

    def row_loop(r, c):
        process_row(wid * rows_per_worker + r)
        return c
    lax.fori_loop(0, rows_per_worker, row_loop, 0)


def _sc_kwinner(x, row_start, sc_rows):
    b, n = x.shape
    k = int(n * DENSITY)
    mesh = plsc.VectorSubcoreMesh(core_axis_name="c", subcore_axis_name="s")
    fn = functools.partial(
        pl.kernel,
        mesh=mesh,
        out_type=jax.ShapeDtypeStruct((sc_rows, n), jnp.float32),
        compiler_params=pltpu.CompilerParams(needs_layout_passes=False),
        scratch_types=[
            pltpu.VMEM((n,), jnp.float32),
            pltpu.VMEM((n,), jnp.int32),
            pltpu.VMEM((2 * 16 * 257,), jnp.int32),
        ],
    )(functools.partial(_sc_body, row_start=row_start, n_rows=sc_rows,
                        n=n, k=k))
    return fn(x)


@jax.jit
def kernel(x):
    b, n = x.shape
    tc_rows = b - SC_ROWS
    out_sc = _sc_kwinner(x, tc_rows, SC_ROWS)
    out_tc = _tc_kwinner(x, tc_rows, full_rows=b)
    return lax.dynamic_update_slice(out_tc, out_sc, (tc_rows, 0))


# speedup vs baseline: 1.1756x; 1.1756x over previous
import functools

import jax
import jax.numpy as jnp
from jax import lax
from jax.experimental import pallas as pl
from jax.experimental.pallas import tpu as pltpu
from jax.experimental.pallas import tpu_sc as plsc

DENSITY = 0.05
SC_ROWS = 32


def _count_cmp(keys16, cand_s16, rows, n, strict):
    n_acc = 8
    accs = [jnp.zeros((rows, 256), jnp.int16) for _ in range(n_acc)]
    for c in range(n // 256):
        blk = keys16[:, c * 256:(c + 1) * 256]
        m = (blk > cand_s16) if strict else (blk >= cand_s16)
        accs[c % n_acc] = accs[c % n_acc] + m.astype(jnp.int16)
    while len(accs) > 1:
        accs = [a + b for a, b in zip(accs[::2], accs[1::2])]
    return jnp.sum(accs[0].astype(jnp.int32), axis=1, keepdims=True)


def _count_ge(keys16, cand_s16, rows, n):
    return _count_cmp(keys16, cand_s16, rows, n, strict=False)


def _kwinner_block(x_ref, o_ref, *, k):
    imin = jnp.int32(-2147483648)
    x = x_ref[...]
    rows, n = x.shape
    i = jax.lax.bitcast_convert_type(x, jnp.int32)
    v = jnp.where(i >= 0, i, jnp.bitwise_xor(jnp.bitwise_not(i), imin))

    hi = jax.lax.shift_right_arithmetic(v, 16).astype(jnp.int16)
    lo = jnp.bitwise_xor(v.astype(jnp.int16), jnp.int16(-32768))

    kk = jnp.int32(k)

    def to_s16(cand_u):
        return jnp.bitwise_xor(cand_u, jnp.int32(0x8000)).astype(jnp.int16)

    def body1(j, t_u):
        bit = jnp.left_shift(jnp.int32(1), 15 - j)
        cand_u = jnp.bitwise_or(t_u, bit)
        cnt = _count_ge(hi, to_s16(cand_u), rows, n)
        return jnp.where(cnt >= kk, cand_u, t_u)

    t_hi_u = jax.lax.fori_loop(0, 16, body1, jnp.zeros((rows, 1), jnp.int32))
    t_hi_s = to_s16(t_hi_u)

    tie = hi == t_hi_s
    c_gt = _count_cmp(hi, t_hi_s, rows, n, strict=True)
    k2 = kk - c_gt

    mlo = jnp.where(tie, lo, jnp.int16(-32768))

    def body2(j, t_u):
        bit = jnp.left_shift(jnp.int32(1), 15 - j)
        cand_u = jnp.bitwise_or(t_u, bit)
        cnt = _count_ge(mlo, to_s16(cand_u), rows, n)
        return jnp.where(cnt >= k2, cand_u, t_u)

    t_lo_u = jax.lax.fori_loop(0, 16, body2, jnp.zeros((rows, 1), jnp.int32))
    t_lo_s = to_s16(t_lo_u)

    keep = jnp.logical_or(hi > t_hi_s, jnp.logical_and(tie, lo >= t_lo_s))
    o_ref[...] = jnp.where(keep, x, 0.0)


def _tc_kwinner(x, tc_rows, full_rows=None):
    b, n = x.shape
    k = int(n * DENSITY)
    rows_per_block = 8
    grid = (tc_rows // rows_per_block,)
    out_rows = full_rows if full_rows is not None else tc_rows
    return pl.pallas_call(
        functools.partial(_kwinner_block, k=k),
        grid=grid,
        in_specs=[pl.BlockSpec((rows_per_block, n), lambda i: (i, 0))],
        out_specs=pl.BlockSpec((rows_per_block, n), lambda i: (i, 0)),
        out_shape=jax.ShapeDtypeStruct((out_rows, n), x.dtype),
        compiler_params=pltpu.CompilerParams(
            dimension_semantics=("parallel",)),
    )(x)


def _sc_body(x_hbm, out_hbm, xrow, key, hist, *, row_start, n_rows, n, k):
    nc = 2
    wid = lax.axis_index("s") * nc + lax.axis_index("c")
    rows_per_worker = n_rows // 32
    chunks = n // 16
    k_cap = jnp.int32(k)

    imin = jnp.int32(-2147483648)
    lane = lax.iota(jnp.int32, 16)
    lane_base = lane * 257
    ones16 = jnp.ones((16,), jnp.int32)
    zeros16 = jnp.zeros((16,), jnp.int32)

    def zero_hist():
        @plsc.parallel_loop(0, 257)
        def _(j):
            hist[pl.ds(j * 16, 16)] = zeros16

    def process_row(row):
        pltpu.sync_copy(x_hbm.at[row_start + row], xrow)

        c_above = jnp.int32(0)
        prefix = jnp.int32(0)

        for level in range(4):
            shift = 24 - 8 * level
            zero_hist()

            if level == 0:
                @plsc.parallel_loop(0, chunks, unroll=8)
                def _(i):
                    xv = xrow[pl.ds(i * 16, 16)]
                    iv = lax.bitcast_convert_type(xv, jnp.int32)
                    kv = jnp.where(iv >= 0,
                                   jnp.bitwise_xor(iv, imin),
                                   jnp.bitwise_not(iv))
                    key[pl.ds(i * 16, 16)] = kv
                    idx = lax.shift_right_logical(
                        kv, jnp.int32(24)) + lane_base
                    plsc.addupdate_scatter(hist, [idx], ones16)
            else:
                @plsc.parallel_loop(0, chunks, unroll=8)
                def _(i, shift=shift, prefix=prefix):
                    kv = key[pl.ds(i * 16, 16)]
                    digit = jnp.bitwise_and(
                        lax.shift_right_logical(kv, jnp.int32(shift)),
                        jnp.int32(0xFF))
                    act = lax.shift_right_logical(
                        kv, jnp.int32(shift + 8)) == prefix
                    idx = jnp.where(act, digit, jnp.int32(256)) + lane_base
                    plsc.addupdate_scatter(hist, [idx], ones16)

            group_counts = []
            for g in range(16):
                acc = hist[pl.ds(g * 16, 16)]
                for l in range(1, 16):
                    acc = acc + hist[pl.ds(l * 257 + g * 16, 16)]
                group_counts.append(acc)
            totals = [jnp.sum(cg) for cg in group_counts]

            found = jnp.int32(0)
            gsel = jnp.int32(0)
            above_at_sel = c_above
            run = c_above
            for g in reversed(range(16)):
                newrun = run + totals[g]
                hit = jnp.logical_and(newrun >= k_cap, found == 0)
                gsel = jnp.where(hit, jnp.int32(g), gsel)
                above_at_sel = jnp.where(hit, run, above_at_sel)
                found = jnp.where(hit, jnp.int32(1), found)
                run = newrun

            csel = group_counts[0]
            for g in range(1, 16):
                csel = jnp.where(gsel == g, group_counts[g], csel)

            suf = lax.rev(jnp.cumsum(lax.rev(csel, (0,))), (0,))
            qual = (above_at_sel + suf) >= k_cap
            n_true = jnp.sum(qual.astype(jnp.int32))
            byte = n_true - 1
            t_byte = gsel * 16 + byte
            c_above = above_at_sel + jnp.sum(
                jnp.where(lane > byte, csel, 0))
            prefix = lax.shift_left(prefix, jnp.int32(8)) + t_byte

        t_s = jnp.bitwise_xor(prefix, imin)

        @plsc.parallel_loop(0, chunks, unroll=8)
        def _(i):
            kv = key[pl.ds(i * 16, 16)]
            ks = jnp.bitwise_xor(kv, imin)
            xv = xrow[pl.ds(i * 16, 16)]
            xrow[pl.ds(i * 16, 16)] = jnp.where(ks >= t_s, xv, 0.0)

        pltpu.sync_copy(xrow, out_hbm.at[row])

    def row_loop(r, c):
        process_row(wid * rows_per_worker + r)
        return c
    lax.fori_loop(0, rows_per_worker, row_loop, 0)


def _sc_kwinner(x, row_start, sc_rows):
    b, n = x.shape
    k = int(n * DENSITY)
    mesh = plsc.VectorSubcoreMesh(core_axis_name="c", subcore_axis_name="s")
    fn = functools.partial(
        pl.kernel,
        mesh=mesh,
        out_type=jax.ShapeDtypeStruct((sc_rows, n), jnp.float32),
        compiler_params=pltpu.CompilerParams(needs_layout_passes=False),
        scratch_types=[
            pltpu.VMEM((n,), jnp.float32),
            pltpu.VMEM((n,), jnp.int32),
            pltpu.VMEM((16 * 257,), jnp.int32),
        ],
    )(functools.partial(_sc_body, row_start=row_start, n_rows=sc_rows,
                        n=n, k=k))
    return fn(x)


@jax.jit
def kernel(x):
    b, n = x.shape
    tc_rows = b - SC_ROWS
    out_sc = _sc_kwinner(x, tc_rows, SC_ROWS)
    out_tc = _tc_kwinner(x, tc_rows, full_rows=b)
    return lax.dynamic_update_slice(out_tc, out_sc, (tc_rows, 0))


# speedup vs baseline: 1.5479x; 1.3167x over previous
import functools

import jax
import jax.numpy as jnp
from jax import lax
from jax.experimental import pallas as pl
from jax.experimental.pallas import tpu as pltpu
from jax.experimental.pallas import tpu_sc as plsc

DENSITY = 0.05
SC_ROWS = 64


def _count_cmp(keys16, cand_s16, rows, n, strict):
    n_acc = 8
    accs = [jnp.zeros((rows, 256), jnp.int16) for _ in range(n_acc)]
    for c in range(n // 256):
        blk = keys16[:, c * 256:(c + 1) * 256]
        m = (blk > cand_s16) if strict else (blk >= cand_s16)
        accs[c % n_acc] = accs[c % n_acc] + m.astype(jnp.int16)
    while len(accs) > 1:
        accs = [a + b for a, b in zip(accs[::2], accs[1::2])]
    return jnp.sum(accs[0].astype(jnp.int32), axis=1, keepdims=True)


def _count_ge(keys16, cand_s16, rows, n):
    return _count_cmp(keys16, cand_s16, rows, n, strict=False)


def _kwinner_block(x_ref, o_ref, *, k):
    imin = jnp.int32(-2147483648)
    x = x_ref[...]
    rows, n = x.shape
    i = jax.lax.bitcast_convert_type(x, jnp.int32)
    v = jnp.where(i >= 0, i, jnp.bitwise_xor(jnp.bitwise_not(i), imin))

    hi = jax.lax.shift_right_arithmetic(v, 16).astype(jnp.int16)
    lo = jnp.bitwise_xor(v.astype(jnp.int16), jnp.int16(-32768))

    kk = jnp.int32(k)

    def to_s16(cand_u):
        return jnp.bitwise_xor(cand_u, jnp.int32(0x8000)).astype(jnp.int16)

    def body1(j, t_u):
        bit = jnp.left_shift(jnp.int32(1), 15 - j)
        cand_u = jnp.bitwise_or(t_u, bit)
        cnt = _count_ge(hi, to_s16(cand_u), rows, n)
        return jnp.where(cnt >= kk, cand_u, t_u)

    t_hi_u = jax.lax.fori_loop(0, 16, body1, jnp.zeros((rows, 1), jnp.int32))
    t_hi_s = to_s16(t_hi_u)

    tie = hi == t_hi_s
    c_gt = _count_cmp(hi, t_hi_s, rows, n, strict=True)
    k2 = kk - c_gt

    mlo = jnp.where(tie, lo, jnp.int16(-32768))

    def body2(j, t_u):
        bit = jnp.left_shift(jnp.int32(1), 15 - j)
        cand_u = jnp.bitwise_or(t_u, bit)
        cnt = _count_ge(mlo, to_s16(cand_u), rows, n)
        return jnp.where(cnt >= k2, cand_u, t_u)

    t_lo_u = jax.lax.fori_loop(0, 16, body2, jnp.zeros((rows, 1), jnp.int32))
    t_lo_s = to_s16(t_lo_u)

    keep = jnp.logical_or(hi > t_hi_s, jnp.logical_and(tie, lo >= t_lo_s))
    o_ref[...] = jnp.where(keep, x, 0.0)


def _tc_kwinner(x, tc_rows, full_rows=None):
    b, n = x.shape
    k = int(n * DENSITY)
    rows_per_block = 8
    grid = (tc_rows // rows_per_block,)
    out_rows = full_rows if full_rows is not None else tc_rows
    return pl.pallas_call(
        functools.partial(_kwinner_block, k=k),
        grid=grid,
        in_specs=[pl.BlockSpec((rows_per_block, n), lambda i: (i, 0))],
        out_specs=pl.BlockSpec((rows_per_block, n), lambda i: (i, 0)),
        out_shape=jax.ShapeDtypeStruct((out_rows, n), x.dtype),
        compiler_params=pltpu.CompilerParams(
            dimension_semantics=("parallel",)),
    )(x)


def _sc_body(x_hbm, out_hbm, xrow, key, hist, *, row_start, n_rows, n, k):
    nc = 2
    wid = lax.axis_index("s") * nc + lax.axis_index("c")
    rows_per_worker = n_rows // 32
    chunks = n // 16
    k_cap = jnp.int32(k)

    imin = jnp.int32(-2147483648)
    lane = lax.iota(jnp.int32, 16)
    lane_base = lane * 257
    ones16 = jnp.ones((16,), jnp.int32)
    zeros16 = jnp.zeros((16,), jnp.int32)

    def zero_hist():
        @plsc.parallel_loop(0, 257)
        def _(j):
            hist[pl.ds(j * 16, 16)] = zeros16

    def process_row(row):
        pltpu.sync_copy(x_hbm.at[row_start + row], xrow)

        c_above = jnp.int32(0)
        prefix = jnp.int32(0)

        for level in range(4):
            shift = 24 - 8 * level
            zero_hist()

            if level == 0:
                @plsc.parallel_loop(0, chunks, unroll=8)
                def _(i):
                    xv = xrow[pl.ds(i * 16, 16)]
                    iv = lax.bitcast_convert_type(xv, jnp.int32)
                    kv = jnp.where(iv >= 0,
                                   jnp.bitwise_xor(iv, imin),
                                   jnp.bitwise_not(iv))
                    key[pl.ds(i * 16, 16)] = kv
                    idx = lax.shift_right_logical(
                        kv, jnp.int32(24)) + lane_base
                    plsc.addupdate_scatter(hist, [idx], ones16)
            else:
                @plsc.parallel_loop(0, chunks, unroll=8)
                def _(i, shift=shift, prefix=prefix):
                    kv = key[pl.ds(i * 16, 16)]
                    digit = jnp.bitwise_and(
                        lax.shift_right_logical(kv, jnp.int32(shift)),
                        jnp.int32(0xFF))
                    act = lax.shift_right_logical(
                        kv, jnp.int32(shift + 8)) == prefix
                    idx = jnp.where(act, digit, jnp.int32(256)) + lane_base
                    plsc.addupdate_scatter(hist, [idx], ones16)

            group_counts = []
            for g in range(16):
                acc = hist[pl.ds(g * 16, 16)]
                for l in range(1, 16):
                    acc = acc + hist[pl.ds(l * 257 + g * 16, 16)]
                group_counts.append(acc)
            totals = [jnp.sum(cg) for cg in group_counts]

            found = jnp.int32(0)
            gsel = jnp.int32(0)
            above_at_sel = c_above
            run = c_above
            for g in reversed(range(16)):
                newrun = run + totals[g]
                hit = jnp.logical_and(newrun >= k_cap, found == 0)
                gsel = jnp.where(hit, jnp.int32(g), gsel)
                above_at_sel = jnp.where(hit, run, above_at_sel)
                found = jnp.where(hit, jnp.int32(1), found)
                run = newrun

            csel = group_counts[0]
            for g in range(1, 16):
                csel = jnp.where(gsel == g, group_counts[g], csel)

            suf = lax.rev(jnp.cumsum(lax.rev(csel, (0,))), (0,))
            qual = (above_at_sel + suf) >= k_cap
            n_true = jnp.sum(qual.astype(jnp.int32))
            byte = n_true - 1
            t_byte = gsel * 16 + byte
            c_above = above_at_sel + jnp.sum(
                jnp.where(lane > byte, csel, 0))
            prefix = lax.shift_left(prefix, jnp.int32(8)) + t_byte

        t_s = jnp.bitwise_xor(prefix, imin)

        @plsc.parallel_loop(0, chunks, unroll=8)
        def _(i):
            kv = key[pl.ds(i * 16, 16)]
            ks = jnp.bitwise_xor(kv, imin)
            xv = xrow[pl.ds(i * 16, 16)]
            xrow[pl.ds(i * 16, 16)] = jnp.where(ks >= t_s, xv, 0.0)

        pltpu.sync_copy(xrow, out_hbm.at[row])

    def row_loop(r, c):
        process_row(wid * rows_per_worker + r)
        return c
    lax.fori_loop(0, rows_per_worker, row_loop, 0)


def _sc_kwinner(x, row_start, sc_rows):
    b, n = x.shape
    k = int(n * DENSITY)
    mesh = plsc.VectorSubcoreMesh(core_axis_name="c", subcore_axis_name="s")
    fn = functools.partial(
        pl.kernel,
        mesh=mesh,
        out_type=jax.ShapeDtypeStruct((sc_rows, n), jnp.float32),
        compiler_params=pltpu.CompilerParams(needs_layout_passes=False),
        scratch_types=[
            pltpu.VMEM((n,), jnp.float32),
            pltpu.VMEM((n,), jnp.int32),
            pltpu.VMEM((16 * 257,), jnp.int32),
        ],
    )(functools.partial(_sc_body, row_start=row_start, n_rows=sc_rows,
                        n=n, k=k))
    return fn(x)


@jax.jit
def kernel(x):
    b, n = x.shape
    tc_rows = b - SC_ROWS
    out_sc = _sc_kwinner(x, tc_rows, SC_ROWS)
    out_tc = _tc_kwinner(x, tc_rows, full_rows=b)
    return lax.dynamic_update_slice(out_tc, out_sc, (tc_rows, 0))
